# baseline (speedup 1.0000x reference)
import functools

import jax
import jax.numpy as jnp
from jax import lax
from jax.experimental import pallas as pl
from jax.experimental.pallas import tpu as pltpu
from jax.experimental.pallas import tpu_sc as plsc

N = 10000
D = 128
E = 320000
NC = 2
NS = 16
NW = NC * NS
CH = 64
NPAD = 10240
NT = 10112
EPAD = 327680
EW = EPAD // NW
NCHUNK = EW // CH
TOTCH = EPAD // CH
S = NPAD // NS

_f32 = jnp.float32
_i32 = jnp.int32


def _tc_pre_body(x_ref, wt_ref, att8_ref, xp_ref, a8_ref):
    xb = x_ref[...]
    xp = jnp.dot(xb, wt_ref[...], preferred_element_type=_f32)
    xp_ref[...] = xp
    a8_ref[...] = lax.dot_general(
        att8_ref[...], xp, (((1,), (1,)), ((), ())),
        preferred_element_type=_f32)


def _tc_prologue(x_pad, wt, att8):
    br = 1024
    return pl.pallas_call(
        _tc_pre_body,
        grid=(NPAD // br,),
        in_specs=[
            pl.BlockSpec((br, D), lambda i: (i, 0)),
            pl.BlockSpec((D, D), lambda i: (0, 0)),
            pl.BlockSpec((8, D), lambda i: (0, 0)),
        ],
        out_specs=[
            pl.BlockSpec((br, D), lambda i: (i, 0)),
            pl.BlockSpec((8, br), lambda i: (0, i)),
        ],
        out_shape=[
            jax.ShapeDtypeStruct((NPAD, D), _f32),
            jax.ShapeDtypeStruct((8, NPAD), _f32),
        ],
    )(x_pad, wt, att8)


def _tc_mid_body(acc0_ref, acc1_ref, den_ref, brow_ref, wt_ref,
                 att8_ref, pf_ref, xp_ref, a8_ref):
    i = pl.program_id(0)
    d = jnp.sum(den_ref[...], axis=0) + 1e-16
    s = (acc0_ref[0] + acc1_ref[0]) / d[:, None] + brow_ref[...]
    rowid = lax.broadcasted_iota(_i32, s.shape, 0) + i * s.shape[0]
    s = jnp.where(rowid < N, s, 0.0)
    pf_ref[...] = s
    xp = jnp.dot(s, wt_ref[...], preferred_element_type=_f32)
    xp_ref[...] = xp
    a8_ref[...] = lax.dot_general(
        att8_ref[...], xp, (((1,), (1,)), ((), ())),
        preferred_element_type=_f32)


def _tc_mid(accs, dens, brow, wt, att8):
    br = 1024
    return pl.pallas_call(
        _tc_mid_body,
        grid=(NPAD // br,),
        in_specs=[
            pl.BlockSpec((1, br, D), lambda i: (0, i, 0)),
            pl.BlockSpec((1, br, D), lambda i: (1, i, 0)),
            pl.BlockSpec((NW, br), lambda i: (0, i)),
            pl.BlockSpec((1, D), lambda i: (0, 0)),
            pl.BlockSpec((D, D), lambda i: (0, 0)),
            pl.BlockSpec((8, D), lambda i: (0, 0)),
        ],
        out_specs=[
            pl.BlockSpec((br, D), lambda i: (i, 0)),
            pl.BlockSpec((br, D), lambda i: (i, 0)),
            pl.BlockSpec((8, br), lambda i: (0, i)),
        ],
        out_shape=[
            jax.ShapeDtypeStruct((NPAD, D), _f32),
            jax.ShapeDtypeStruct((NPAD, D), _f32),
            jax.ShapeDtypeStruct((8, NPAD), _f32),
        ],
    )(accs, accs, dens, brow, wt, att8)


def _tc_final_body(acc0_ref, acc1_ref, den_ref, brow_ref, out_ref):
    d = jnp.sum(den_ref[...], axis=0) + 1e-16
    out_ref[...] = (acc0_ref[0] + acc1_ref[0]) / d[:, None] + brow_ref[...]


def _tc_final(accs, dens, brow):
    br = 1024
    return pl.pallas_call(
        _tc_final_body,
        grid=(NPAD // br,),
        in_specs=[
            pl.BlockSpec((1, br, D), lambda i: (0, i, 0)),
            pl.BlockSpec((1, br, D), lambda i: (1, i, 0)),
            pl.BlockSpec((NW, br), lambda i: (0, i)),
            pl.BlockSpec((1, D), lambda i: (0, 0)),
        ],
        out_specs=pl.BlockSpec((br, D), lambda i: (i, 0)),
        out_shape=jax.ShapeDtypeStruct((NPAD, D), _f32),
    )(accs, accs, dens, brow)


_MESH = plsc.VectorSubcoreMesh(
    core_axis_name="c", subcore_axis_name="s", num_cores=NC, num_subcores=NS)

_SC_PARAMS = pltpu.CompilerParams(needs_layout_passes=False)


@functools.partial(
    pl.kernel,
    mesh=_MESH,
    compiler_params=_SC_PARAMS,
    out_type=[
        jax.ShapeDtypeStruct((NC, NPAD, D), _f32),
        jax.ShapeDtypeStruct((NW, NPAD), _f32),
    ],
    scratch_types=[
        pltpu.VMEM((NT,), _f32),
        pltpu.VMEM((NT,), _f32),
        pltpu.VMEM((NPAD,), _f32),
        pltpu.VMEM((4, 2, CH), _i32),
        pltpu.VMEM((4, 2, CH), _i32),
        pltpu.VMEM((CH,), _f32),
        pltpu.VMEM((CH, D), _f32),
        pltpu.VMEM((CH, D), _f32),
        pltpu.VMEM_SHARED((NPAD, D), _f32),
        pltpu.SemaphoreType.DMA,
        pltpu.SemaphoreType.DMA,
        pltpu.SemaphoreType.DMA,
        pltpu.SemaphoreType.DMA,
        pltpu.SemaphoreType.DMA,
        pltpu.SemaphoreType.DMA,
        pltpu.SemaphoreType.DMA,
    ],
)
def _sc_edge(xp_hbm, ep_hbm, a8_hbm, acc_hbm, den_hbm,
             asrc_t, adst_t, den_t, ebb0, ebb1, exb, rw0, rw1,
             accsh, isem0, isem1, gsem0, gsem1, ssem0, ssem1, zsem):
    cid = lax.axis_index("c")
    sid = lax.axis_index("s")
    wid = sid * NC + cid
    ebbs = (ebb0, ebb1)
    rws = (rw0, rw1)
    isems = (isem0, isem1)
    gsems = (gsem0, gsem1)
    ssems = (ssem0, ssem1)
    NQ = NCHUNK // 4

    zero16 = jnp.zeros((16,), _f32)

    def zrow(r, _):
        for k in range(D // 16):
            rw0[r, pl.ds(k * 16, 16)] = zero16
        return 0
    lax.fori_loop(0, CH, zrow, 0)
    for t in range(S // CH):
        pltpu.async_copy(rw0, accsh.at[pl.ds(sid * S + t * CH, CH)], zsem)

    pltpu.async_copy(ep_hbm.at[wid, 0], ebb0, isem0)
    pltpu.async_copy(ep_hbm.at[wid, 1], ebb1, isem1)
    pltpu.sync_copy(a8_hbm.at[0, pl.ds(0, NT)], asrc_t)
    pltpu.sync_copy(a8_hbm.at[1, pl.ds(0, NT)], adst_t)

    def zbody(i, _):
        den_t[pl.ds(i * 16, 16)] = zero16
        return 0
    lax.fori_loop(0, NPAD // 16, zbody, 0)

    for t in range(S // CH):
        pltpu.make_async_copy(
            rw0, accsh.at[pl.ds(sid * S + t * CH, CH)], zsem).wait()
    plsc.subcore_barrier()

    pltpu.make_async_copy(ep_hbm.at[wid, 0], ebb0, isem0).wait()
    pltpu.async_copy(xp_hbm.at[ebb0.at[0, 0]], rw0, gsem0)

    def outer(qq, _):
        for p in range(2):
            qsuper = qq * 2 + p
            ebb = ebbs[p]
            for u in range(4):
                j = qsuper * 4 + u
                b2 = u % 2
                rw = rws[b2]

                pltpu.make_async_copy(
                    xp_hbm.at[ebb.at[u, 0]], rw, gsems[b2]).wait()

                for k in range(CH // 16):
                    sl = pl.ds(k * 16, 16)
                    sv = ebb[u, 0, sl]
                    dv = ebb[u, 1, sl]
                    av = plsc.load_gather(asrc_t, [sv])
                    bv = plsc.load_gather(adst_t, [dv])
                    al = av + bv
                    al = jnp.where(al >= 0.0, al, al * 0.2)
                    ev = jnp.exp(al)
                    exb[sl] = ev
                    plsc.addupdate_scatter(den_t, [dv], ev)

                @pl.when(j + 1 < NCHUNK)
                def _():
                    @pl.when(j >= 1)
                    def _():
                        if u == 0:
                            eprev = ebbs[1 - p].at[3, 1]
                        else:
                            eprev = ebb.at[u - 1, 1]
                        pltpu.make_async_copy(
                            rws[1 - b2], accsh.at[eprev],
                            ssems[1 - b2]).wait()
                        if u == 0:
                            @pl.when(qsuper + 1 < NQ)
                            def _():
                                pltpu.async_copy(
                                    ep_hbm.at[wid, qsuper + 1],
                                    ebbs[1 - p], isems[1 - p])
                    if u == 3:
                        pltpu.make_async_copy(
                            ep_hbm.at[wid, qsuper + 1], ebbs[1 - p],
                            isems[1 - p]).wait()
                        enext = ebbs[1 - p].at[0, 0]
                    else:
                        enext = ebb.at[u + 1, 0]
                    pltpu.async_copy(
                        xp_hbm.at[enext], rws[1 - b2], gsems[1 - b2])

                def scale(g, _):
                    for l in range(16):
                        r = g * 16 + l
                        cvec = plsc.load_gather(
                            exb, [jnp.full((16,), r, _i32)])
                        for k in range(D // 16):
                            csl = pl.ds(k * 16, 16)
                            rw[r, csl] = rw[r, csl] * cvec
                    return 0
                lax.fori_loop(0, CH // 16, scale, 0)

                pltpu.async_copy(rw, accsh.at[ebb.at[u, 1]], ssems[b2],
                                 add=True)
        return 0
    lax.fori_loop(0, NQ // 2, outer, 0)

    _eblast = ebbs[(NQ - 1) % 2]
    pltpu.make_async_copy(rws[0], accsh.at[_eblast.at[2, 1]], ssems[0]).wait()
    pltpu.make_async_copy(rws[1], accsh.at[_eblast.at[3, 1]], ssems[1]).wait()

    pltpu.sync_copy(den_t, den_hbm.at[wid])
    plsc.subcore_barrier()

    for t in range(S // CH):
        r0 = sid * S + t * CH
        pltpu.async_copy(
            accsh.at[pl.ds(r0, CH)], acc_hbm.at[cid, pl.ds(r0, CH)], zsem)
    for t in range(S // CH):
        r0 = sid * S + t * CH
        pltpu.make_async_copy(
            accsh.at[pl.ds(r0, CH)], acc_hbm.at[cid, pl.ds(r0, CH)],
            zsem).wait()


def _pack_att8(att_src, att_dst):
    return jnp.concatenate(
        [att_src[None, :], att_dst[None, :], jnp.zeros((6, D), _f32)], axis=0)


def _pack_edges(edge_index):
    src = edge_index[0].astype(_i32)
    dst = edge_index[1].astype(_i32)
    pad = EPAD - E
    src = jnp.concatenate([src, jnp.arange(pad, dtype=_i32) % N])
    dst = jnp.concatenate([dst, jnp.full((pad,), N, _i32)])
    nq = NCHUNK // 4
    return jnp.stack(
        [src.reshape(NW, nq, 4, CH), dst.reshape(NW, nq, 4, CH)], axis=3)


def kernel(x_author, edge_index_author_to_paper, edge_index_paper_to_author,
           W1, att_src1, att_dst1, b1, W2, att_src2, att_dst2, b2):
    x_pad = jnp.concatenate([x_author, jnp.zeros((NPAD - N, D), _f32)])
    ep1 = _pack_edges(edge_index_author_to_paper)
    ep2 = _pack_edges(edge_index_paper_to_author)

    xp1, a8_1 = _tc_prologue(x_pad, W1.T, _pack_att8(att_src1, att_dst1))
    acc1, den1 = _sc_edge(xp1, ep1, a8_1)

    pf, xp2, a8_2 = _tc_mid(acc1, den1, b1[None, :], W2.T,
                            _pack_att8(att_src2, att_dst2))
    acc2, den2 = _sc_edge(xp2, ep2, a8_2)

    author_feats = _tc_final(acc2, den2, b2[None, :])[:N]
    paper_feats = pf[:N]
    return (author_feats, paper_feats)


# speedup vs baseline: 1.0314x; 1.0314x over previous
import functools

import jax
import jax.numpy as jnp
from jax import lax
from jax.experimental import pallas as pl
from jax.experimental.pallas import tpu as pltpu
from jax.experimental.pallas import tpu_sc as plsc

N = 10000
D = 128
E = 320000
NC = 2
NS = 16
NW = NC * NS
CH = 64
NPAD = 10240
NT = 10112
EPAD = 327680
EW = EPAD // NW
NCHUNK = EW // CH
TOTCH = EPAD // CH
S = NPAD // NS

_f32 = jnp.float32
_i32 = jnp.int32


def _tc_pre_body(x_ref, wt_ref, att8_ref, xp_ref, a8_ref):
    xb = x_ref[...]
    xp = jnp.dot(xb, wt_ref[...], preferred_element_type=_f32)
    xp_ref[...] = xp
    a8_ref[...] = lax.dot_general(
        att8_ref[...], xp, (((1,), (1,)), ((), ())),
        preferred_element_type=_f32)


def _tc_prologue(x_pad, wt, att8):
    br = 1024
    return pl.pallas_call(
        _tc_pre_body,
        grid=(NPAD // br,),
        in_specs=[
            pl.BlockSpec((br, D), lambda i: (i, 0)),
            pl.BlockSpec((D, D), lambda i: (0, 0)),
            pl.BlockSpec((8, D), lambda i: (0, 0)),
        ],
        out_specs=[
            pl.BlockSpec((br, D), lambda i: (i, 0)),
            pl.BlockSpec((8, br), lambda i: (0, i)),
        ],
        out_shape=[
            jax.ShapeDtypeStruct((NPAD, D), _f32),
            jax.ShapeDtypeStruct((8, NPAD), _f32),
        ],
    )(x_pad, wt, att8)


def _tc_mid_body(acc0_ref, acc1_ref, den_ref, brow_ref, wt_ref,
                 att8_ref, pf_ref, xp_ref, a8_ref):
    i = pl.program_id(0)
    d = jnp.sum(den_ref[...], axis=0) + 1e-16
    s = (acc0_ref[0] + acc1_ref[0]) / d[:, None] + brow_ref[...]
    rowid = lax.broadcasted_iota(_i32, s.shape, 0) + i * s.shape[0]
    s = jnp.where(rowid < N, s, 0.0)
    pf_ref[...] = s
    xp = jnp.dot(s, wt_ref[...], preferred_element_type=_f32)
    xp_ref[...] = xp
    a8_ref[...] = lax.dot_general(
        att8_ref[...], xp, (((1,), (1,)), ((), ())),
        preferred_element_type=_f32)


def _tc_mid(accs, dens, brow, wt, att8):
    br = 1024
    return pl.pallas_call(
        _tc_mid_body,
        grid=(NPAD // br,),
        in_specs=[
            pl.BlockSpec((1, br, D), lambda i: (0, i, 0)),
            pl.BlockSpec((1, br, D), lambda i: (1, i, 0)),
            pl.BlockSpec((NW, br), lambda i: (0, i)),
            pl.BlockSpec((1, D), lambda i: (0, 0)),
            pl.BlockSpec((D, D), lambda i: (0, 0)),
            pl.BlockSpec((8, D), lambda i: (0, 0)),
        ],
        out_specs=[
            pl.BlockSpec((br, D), lambda i: (i, 0)),
            pl.BlockSpec((br, D), lambda i: (i, 0)),
            pl.BlockSpec((8, br), lambda i: (0, i)),
        ],
        out_shape=[
            jax.ShapeDtypeStruct((NPAD, D), _f32),
            jax.ShapeDtypeStruct((NPAD, D), _f32),
            jax.ShapeDtypeStruct((8, NPAD), _f32),
        ],
    )(accs, accs, dens, brow, wt, att8)


def _tc_final_body(acc0_ref, acc1_ref, den_ref, brow_ref, out_ref):
    d = jnp.sum(den_ref[...], axis=0) + 1e-16
    out_ref[...] = (acc0_ref[0] + acc1_ref[0]) / d[:, None] + brow_ref[...]


def _tc_final(accs, dens, brow):
    br = 1024
    return pl.pallas_call(
        _tc_final_body,
        grid=(NPAD // br,),
        in_specs=[
            pl.BlockSpec((1, br, D), lambda i: (0, i, 0)),
            pl.BlockSpec((1, br, D), lambda i: (1, i, 0)),
            pl.BlockSpec((NW, br), lambda i: (0, i)),
            pl.BlockSpec((1, D), lambda i: (0, 0)),
        ],
        out_specs=pl.BlockSpec((br, D), lambda i: (i, 0)),
        out_shape=jax.ShapeDtypeStruct((NPAD, D), _f32),
    )(accs, accs, dens, brow)


_MESH = plsc.VectorSubcoreMesh(
    core_axis_name="c", subcore_axis_name="s", num_cores=NC, num_subcores=NS)

_SC_PARAMS = pltpu.CompilerParams(needs_layout_passes=False)


@functools.partial(
    pl.kernel,
    mesh=_MESH,
    compiler_params=_SC_PARAMS,
    out_type=[
        jax.ShapeDtypeStruct((NC, NPAD, D), _f32),
        jax.ShapeDtypeStruct((NW, NPAD), _f32),
    ],
    scratch_types=[
        pltpu.VMEM((NT,), _f32),
        pltpu.VMEM((NT,), _f32),
        pltpu.VMEM((NPAD,), _f32),
        pltpu.VMEM((2, CH), _i32),
        pltpu.VMEM((2, CH), _i32),
        pltpu.VMEM((2, CH), _i32),
        pltpu.VMEM((2, CH), _i32),
        pltpu.VMEM((CH,), _f32),
        pltpu.VMEM((CH, D), _f32),
        pltpu.VMEM((CH, D), _f32),
        pltpu.VMEM_SHARED((NPAD, D), _f32),
        pltpu.SemaphoreType.DMA,
        pltpu.SemaphoreType.DMA,
        pltpu.SemaphoreType.DMA,
        pltpu.SemaphoreType.DMA,
        pltpu.SemaphoreType.DMA,
        pltpu.SemaphoreType.DMA,
        pltpu.SemaphoreType.DMA,
        pltpu.SemaphoreType.DMA,
    ],
)
def _sc_edge(xp_hbm, ep_hbm, a8_hbm, acc_hbm, den_hbm,
             asrc_t, adst_t, den_t, eb0, eb1, eb2, eb3, exb, rw0, rw1,
             accsh, isem0, isem1, isem2, isem3, gsem0, gsem1, ssem0, ssem1):
    cid = lax.axis_index("c")
    sid = lax.axis_index("s")
    wid = sid * NC + cid
    ebs = (eb0, eb1, eb2, eb3)
    rws = (rw0, rw1)
    isems = (isem0, isem1, isem2, isem3)
    gsems = (gsem0, gsem1)
    ssems = (ssem0, ssem1)

    pltpu.sync_copy(a8_hbm.at[0, pl.ds(0, NT)], asrc_t)
    pltpu.sync_copy(a8_hbm.at[1, pl.ds(0, NT)], adst_t)

    zero16 = jnp.zeros((16,), _f32)

    def zbody(i, _):
        den_t[pl.ds(i * 16, 16)] = zero16
        return 0
    lax.fori_loop(0, NPAD // 16, zbody, 0)

    def zrow(r, _):
        for k in range(D // 16):
            rw0[r, pl.ds(k * 16, 16)] = zero16
        return 0
    lax.fori_loop(0, CH, zrow, 0)
    for t in range(S // CH):
        pltpu.async_copy(rw0, accsh.at[pl.ds(sid * S + t * CH, CH)], gsem1)
    for t in range(S // CH):
        pltpu.make_async_copy(
            rw0, accsh.at[pl.ds(sid * S + t * CH, CH)], gsem1).wait()
    plsc.subcore_barrier()

    cbase = wid * NCHUNK

    pltpu.async_copy(ep_hbm.at[cbase], eb0, isem0)
    pltpu.async_copy(ep_hbm.at[cbase + 1], eb1, isem1)
    pltpu.make_async_copy(ep_hbm.at[cbase], eb0, isem0).wait()
    pltpu.async_copy(xp_hbm.at[eb0.at[0]], rw0, gsem0)

    def outer(q, _):
        for u in range(4):
            j = q * 4 + u
            b2 = u % 2
            eb = ebs[u]
            rw = rws[b2]

            pltpu.make_async_copy(xp_hbm.at[eb.at[0]], rw, gsems[b2]).wait()

            for k in range(CH // 16):
                sl = pl.ds(k * 16, 16)
                sv = eb[0, sl]
                dv = eb[1, sl]
                av = plsc.load_gather(asrc_t, [sv])
                bv = plsc.load_gather(adst_t, [dv])
                al = av + bv
                al = jnp.where(al >= 0.0, al, al * 0.2)
                ev = jnp.exp(al)
                exb[sl] = ev
                plsc.addupdate_scatter(den_t, [dv], ev)

            @pl.when(j + 1 < NCHUNK)
            def _():
                @pl.when(j >= 1)
                def _():
                    eprev = ebs[(u + 3) % 4]
                    pltpu.make_async_copy(
                        rws[1 - b2], accsh.at[eprev.at[1]],
                        ssems[1 - b2]).wait()
                pltpu.make_async_copy(
                    ep_hbm.at[cbase + j + 1], ebs[(u + 1) % 4],
                    isems[(u + 1) % 4]).wait()
                pltpu.async_copy(
                    xp_hbm.at[ebs[(u + 1) % 4].at[0]], rws[1 - b2],
                    gsems[1 - b2])

            @pl.when(j + 2 < NCHUNK)
            def _():
                pltpu.async_copy(
                    ep_hbm.at[cbase + j + 2], ebs[(u + 2) % 4],
                    isems[(u + 2) % 4])

            def scale(g, _):
                for l in range(16):
                    r = g * 16 + l
                    cvec = plsc.load_gather(exb, [jnp.full((16,), r, _i32)])
                    for k in range(D // 16):
                        csl = pl.ds(k * 16, 16)
                        rw[r, csl] = rw[r, csl] * cvec
                return 0
            lax.fori_loop(0, CH // 16, scale, 0)

            pltpu.async_copy(rw, accsh.at[eb.at[1]], ssems[b2], add=True)
        return 0
    lax.fori_loop(0, NCHUNK // 4, outer, 0)

    pltpu.make_async_copy(
        rws[0], accsh.at[ebs[(NCHUNK - 2) % 4].at[1]], ssems[0]).wait()
    pltpu.make_async_copy(
        rws[1], accsh.at[ebs[(NCHUNK - 1) % 4].at[1]], ssems[1]).wait()

    pltpu.sync_copy(den_t, den_hbm.at[wid])
    plsc.subcore_barrier()

    for t in range(S // CH):
        r0 = sid * S + t * CH
        pltpu.async_copy(
            accsh.at[pl.ds(r0, CH)], acc_hbm.at[cid, pl.ds(r0, CH)], gsem0)
    for t in range(S // CH):
        r0 = sid * S + t * CH
        pltpu.make_async_copy(
            accsh.at[pl.ds(r0, CH)], acc_hbm.at[cid, pl.ds(r0, CH)],
            gsem0).wait()


def _pack_att8(att_src, att_dst):
    return jnp.concatenate(
        [att_src[None, :], att_dst[None, :], jnp.zeros((6, D), _f32)], axis=0)


def _pack_edges(edge_index):
    src = edge_index[0].astype(_i32)
    dst = edge_index[1].astype(_i32)
    pad = EPAD - E
    src = jnp.concatenate([src, jnp.arange(pad, dtype=_i32) % N])
    dst = jnp.concatenate([dst, jnp.full((pad,), N, _i32)])
    return jnp.stack(
        [src.reshape(TOTCH, CH), dst.reshape(TOTCH, CH)], axis=1)


def kernel(x_author, edge_index_author_to_paper, edge_index_paper_to_author,
           W1, att_src1, att_dst1, b1, W2, att_src2, att_dst2, b2):
    x_pad = jnp.concatenate([x_author, jnp.zeros((NPAD - N, D), _f32)])
    ep1 = _pack_edges(edge_index_author_to_paper)
    ep2 = _pack_edges(edge_index_paper_to_author)

    xp1, a8_1 = _tc_prologue(x_pad, W1.T, _pack_att8(att_src1, att_dst1))
    acc1, den1 = _sc_edge(xp1, ep1, a8_1)

    pf, xp2, a8_2 = _tc_mid(acc1, den1, b1[None, :], W2.T,
                            _pack_att8(att_src2, att_dst2))
    acc2, den2 = _sc_edge(xp2, ep2, a8_2)

    author_feats = _tc_final(acc2, den2, b2[None, :])[:N]
    paper_feats = pf[:N]
    return (author_feats, paper_feats)


# speedup vs baseline: 1.0436x; 1.0118x over previous
import functools

import jax
import jax.numpy as jnp
from jax import lax
from jax.experimental import pallas as pl
from jax.experimental.pallas import tpu as pltpu
from jax.experimental.pallas import tpu_sc as plsc

N = 10000
D = 128
E = 320000
NC = 2
NS = 16
NW = NC * NS
CH = 64
NPAD = 10240
NT = 10112
EPAD = 327680
EW = EPAD // NW
NCHUNK = EW // CH
TOTCH = EPAD // CH
S = NPAD // NS

_f32 = jnp.float32
_i32 = jnp.int32


def _tc_pre_body(x_ref, wt_ref, att8_ref, xp_ref, a8_ref):
    xb = x_ref[...]
    xp = jnp.dot(xb, wt_ref[...], preferred_element_type=_f32)
    xp_ref[...] = xp
    a8_ref[...] = lax.dot_general(
        att8_ref[...], xp, (((1,), (1,)), ((), ())),
        preferred_element_type=_f32)


def _tc_prologue(x_pad, wt, att8):
    br = 1024
    return pl.pallas_call(
        _tc_pre_body,
        grid=(NPAD // br,),
        in_specs=[
            pl.BlockSpec((br, D), lambda i: (i, 0)),
            pl.BlockSpec((D, D), lambda i: (0, 0)),
            pl.BlockSpec((8, D), lambda i: (0, 0)),
        ],
        out_specs=[
            pl.BlockSpec((br, D), lambda i: (i, 0)),
            pl.BlockSpec((8, br), lambda i: (0, i)),
        ],
        out_shape=[
            jax.ShapeDtypeStruct((NPAD, D), _f32),
            jax.ShapeDtypeStruct((8, NPAD), _f32),
        ],
    )(x_pad, wt, att8)


def _tc_mid_body(acc0_ref, acc1_ref, den_ref, brow_ref, wt_ref,
                 att8_ref, pf_ref, xp_ref, a8_ref):
    i = pl.program_id(0)
    d = jnp.sum(den_ref[...], axis=0) + 1e-16
    s = (acc0_ref[0] + acc1_ref[0]) / d[:, None] + brow_ref[...]
    rowid = lax.broadcasted_iota(_i32, s.shape, 0) + i * s.shape[0]
    s = jnp.where(rowid < N, s, 0.0)
    pf_ref[...] = s
    xp = jnp.dot(s, wt_ref[...], preferred_element_type=_f32)
    xp_ref[...] = xp
    a8_ref[...] = lax.dot_general(
        att8_ref[...], xp, (((1,), (1,)), ((), ())),
        preferred_element_type=_f32)


def _tc_mid(accs, dens, brow, wt, att8):
    br = 1024
    return pl.pallas_call(
        _tc_mid_body,
        grid=(NPAD // br,),
        in_specs=[
            pl.BlockSpec((1, br, D), lambda i: (0, i, 0)),
            pl.BlockSpec((1, br, D), lambda i: (1, i, 0)),
            pl.BlockSpec((NW, br), lambda i: (0, i)),
            pl.BlockSpec((1, D), lambda i: (0, 0)),
            pl.BlockSpec((D, D), lambda i: (0, 0)),
            pl.BlockSpec((8, D), lambda i: (0, 0)),
        ],
        out_specs=[
            pl.BlockSpec((br, D), lambda i: (i, 0)),
            pl.BlockSpec((br, D), lambda i: (i, 0)),
            pl.BlockSpec((8, br), lambda i: (0, i)),
        ],
        out_shape=[
            jax.ShapeDtypeStruct((NPAD, D), _f32),
            jax.ShapeDtypeStruct((NPAD, D), _f32),
            jax.ShapeDtypeStruct((8, NPAD), _f32),
        ],
    )(accs, accs, dens, brow, wt, att8)


def _tc_final_body(acc0_ref, acc1_ref, den_ref, brow_ref, out_ref):
    d = jnp.sum(den_ref[...], axis=0) + 1e-16
    out_ref[...] = (acc0_ref[0] + acc1_ref[0]) / d[:, None] + brow_ref[...]


def _tc_final(accs, dens, brow):
    br = 1024
    return pl.pallas_call(
        _tc_final_body,
        grid=(NPAD // br,),
        in_specs=[
            pl.BlockSpec((1, br, D), lambda i: (0, i, 0)),
            pl.BlockSpec((1, br, D), lambda i: (1, i, 0)),
            pl.BlockSpec((NW, br), lambda i: (0, i)),
            pl.BlockSpec((1, D), lambda i: (0, 0)),
        ],
        out_specs=pl.BlockSpec((br, D), lambda i: (i, 0)),
        out_shape=jax.ShapeDtypeStruct((NPAD, D), _f32),
    )(accs, accs, dens, brow)


_MESH = plsc.VectorSubcoreMesh(
    core_axis_name="c", subcore_axis_name="s", num_cores=NC, num_subcores=NS)

_SC_PARAMS = pltpu.CompilerParams(needs_layout_passes=False)


@functools.partial(
    pl.kernel,
    mesh=_MESH,
    compiler_params=_SC_PARAMS,
    out_type=[
        jax.ShapeDtypeStruct((NC, NPAD, D), _f32),
        jax.ShapeDtypeStruct((NW, NPAD), _f32),
    ],
    scratch_types=[
        pltpu.VMEM((NT,), _f32),
        pltpu.VMEM((NT,), _f32),
        pltpu.VMEM((NPAD,), _f32),
        pltpu.VMEM((2, CH), _i32),
        pltpu.VMEM((2, CH), _i32),
        pltpu.VMEM((2, CH), _i32),
        pltpu.VMEM((2, CH), _i32),
        pltpu.VMEM((CH,), _f32),
        pltpu.VMEM((CH, D), _f32),
        pltpu.VMEM((CH, D), _f32),
        pltpu.VMEM_SHARED((NPAD, D), _f32),
        pltpu.SemaphoreType.DMA,
        pltpu.SemaphoreType.DMA,
        pltpu.SemaphoreType.DMA,
        pltpu.SemaphoreType.DMA,
        pltpu.SemaphoreType.DMA,
        pltpu.SemaphoreType.DMA,
        pltpu.SemaphoreType.DMA,
        pltpu.SemaphoreType.DMA,
    ],
)
def _sc_edge(xp_hbm, ep_hbm, a8_hbm, acc_hbm, den_hbm,
             asrc_t, adst_t, den_t, eb0, eb1, eb2, eb3, exb, rw0, rw1,
             accsh, isem0, isem1, isem2, isem3, gsem0, gsem1, ssem0, ssem1):
    cid = lax.axis_index("c")
    sid = lax.axis_index("s")
    wid = sid * NC + cid
    ebs = (eb0, eb1, eb2, eb3)
    rws = (rw0, rw1)
    isems = (isem0, isem1, isem2, isem3)
    gsems = (gsem0, gsem1)
    ssems = (ssem0, ssem1)

    pltpu.sync_copy(a8_hbm.at[0, pl.ds(0, NT)], asrc_t)
    pltpu.sync_copy(a8_hbm.at[1, pl.ds(0, NT)], adst_t)

    zero16 = jnp.zeros((16,), _f32)

    def zbody(i, _):
        den_t[pl.ds(i * 16, 16)] = zero16
        return 0
    lax.fori_loop(0, NPAD // 16, zbody, 0)

    def zrow(r, _):
        for k in range(D // 16):
            rw0[r, pl.ds(k * 16, 16)] = zero16
        return 0
    lax.fori_loop(0, CH, zrow, 0)
    for t in range(S // CH):
        pltpu.async_copy(rw0, accsh.at[pl.ds(sid * S + t * CH, CH)], gsem1)
    for t in range(S // CH):
        pltpu.make_async_copy(
            rw0, accsh.at[pl.ds(sid * S + t * CH, CH)], gsem1).wait()
    plsc.subcore_barrier()

    cbase = wid * NCHUNK

    pltpu.async_copy(ep_hbm.at[cbase], eb0, isem0)
    pltpu.async_copy(ep_hbm.at[cbase + 1], eb1, isem1)
    pltpu.make_async_copy(ep_hbm.at[cbase], eb0, isem0).wait()
    pltpu.async_copy(xp_hbm.at[eb0.at[0]], rw0, gsem0)

    def outer(q, _):
        for u in range(4):
            j = q * 4 + u
            b2 = u % 2
            eb = ebs[u]
            rw = rws[b2]

            pltpu.make_async_copy(xp_hbm.at[eb.at[0]], rw, gsems[b2]).wait()

            @pl.when(j + 1 < NCHUNK)
            def _():
                @pl.when(j >= 1)
                def _():
                    eprev = ebs[(u + 3) % 4]
                    pltpu.make_async_copy(
                        rws[1 - b2], accsh.at[eprev.at[1]],
                        ssems[1 - b2]).wait()
                pltpu.make_async_copy(
                    ep_hbm.at[cbase + j + 1], ebs[(u + 1) % 4],
                    isems[(u + 1) % 4]).wait()
                pltpu.async_copy(
                    xp_hbm.at[ebs[(u + 1) % 4].at[0]], rws[1 - b2],
                    gsems[1 - b2])

            for k in range(CH // 16):
                sl = pl.ds(k * 16, 16)
                sv = eb[0, sl]
                dv = eb[1, sl]
                av = plsc.load_gather(asrc_t, [sv])
                bv = plsc.load_gather(adst_t, [dv])
                al = av + bv
                al = jnp.where(al >= 0.0, al, al * 0.2)
                ev = jnp.exp(al)
                exb[sl] = ev
                plsc.addupdate_scatter(den_t, [dv], ev)

            @pl.when(j + 2 < NCHUNK)
            def _():
                pltpu.async_copy(
                    ep_hbm.at[cbase + j + 2], ebs[(u + 2) % 4],
                    isems[(u + 2) % 4])

            def scale(g, _):
                for l in range(16):
                    r = g * 16 + l
                    cvec = plsc.load_gather(exb, [jnp.full((16,), r, _i32)])
                    for k in range(D // 16):
                        csl = pl.ds(k * 16, 16)
                        rw[r, csl] = rw[r, csl] * cvec
                return 0
            lax.fori_loop(0, CH // 16, scale, 0)

            pltpu.async_copy(rw, accsh.at[eb.at[1]], ssems[b2], add=True)
        return 0
    lax.fori_loop(0, NCHUNK // 4, outer, 0)

    pltpu.make_async_copy(
        rws[0], accsh.at[ebs[(NCHUNK - 2) % 4].at[1]], ssems[0]).wait()
    pltpu.make_async_copy(
        rws[1], accsh.at[ebs[(NCHUNK - 1) % 4].at[1]], ssems[1]).wait()

    pltpu.sync_copy(den_t, den_hbm.at[wid])
    plsc.subcore_barrier()

    for t in range(S // CH):
        r0 = sid * S + t * CH
        pltpu.async_copy(
            accsh.at[pl.ds(r0, CH)], acc_hbm.at[cid, pl.ds(r0, CH)], gsem0)
    for t in range(S // CH):
        r0 = sid * S + t * CH
        pltpu.make_async_copy(
            accsh.at[pl.ds(r0, CH)], acc_hbm.at[cid, pl.ds(r0, CH)],
            gsem0).wait()


def _pack_att8(att_src, att_dst):
    return jnp.concatenate(
        [att_src[None, :], att_dst[None, :], jnp.zeros((6, D), _f32)], axis=0)


def _pack_edges(edge_index):
    src = edge_index[0].astype(_i32)
    dst = edge_index[1].astype(_i32)
    pad = EPAD - E
    src = jnp.concatenate([src, jnp.arange(pad, dtype=_i32) % N])
    dst = jnp.concatenate([dst, jnp.full((pad,), N, _i32)])
    return jnp.stack(
        [src.reshape(TOTCH, CH), dst.reshape(TOTCH, CH)], axis=1)


def kernel(x_author, edge_index_author_to_paper, edge_index_paper_to_author,
           W1, att_src1, att_dst1, b1, W2, att_src2, att_dst2, b2):
    x_pad = jnp.concatenate([x_author, jnp.zeros((NPAD - N, D), _f32)])
    ep1 = _pack_edges(edge_index_author_to_paper)
    ep2 = _pack_edges(edge_index_paper_to_author)

    xp1, a8_1 = _tc_prologue(x_pad, W1.T, _pack_att8(att_src1, att_dst1))
    acc1, den1 = _sc_edge(xp1, ep1, a8_1)

    pf, xp2, a8_2 = _tc_mid(acc1, den1, b1[None, :], W2.T,
                            _pack_att8(att_src2, att_dst2))
    acc2, den2 = _sc_edge(xp2, ep2, a8_2)

    author_feats = _tc_final(acc2, den2, b2[None, :])[:N]
    paper_feats = pf[:N]
    return (author_feats, paper_feats)
